# baseline (speedup 1.0000x reference)
import jax
import jax.numpy as jnp
from jax import lax
from jax.experimental import pallas as pl
from jax.experimental.pallas import tpu as pltpu
from jax.experimental.pallas import tpu_sc as plsc

_ROW = 3456
_NW = 32
_B = 128
_NSLOT = 8
_UNROLL = 24
_VEC = 16
_TOTAL = _B * _NW * _ROW


def _add_table(buf, tabv):
    steps = _ROW // (_VEC * _UNROLL)

    def jbody(j, _):
        base = j * (_VEC * _UNROLL)
        for u in range(_UNROLL):
            i = base + u * _VEC
            buf[pl.ds(i, _VEC)] = buf[pl.ds(i, _VEC)] + tabv[pl.ds(i, _VEC)]
        return 0

    lax.fori_loop(0, steps, jbody, 0)


def _sc_body(p_hbm, t_hbm, o_hbm, tabv, *scr):
    bufs = scr[0:_NSLOT]
    gsems = scr[_NSLOT:2 * _NSLOT]
    ssems = scr[2 * _NSLOT:3 * _NSLOT]

    wid = lax.axis_index("s") * 2 + lax.axis_index("c")

    pltpu.sync_copy(t_hbm.at[pl.ds(wid * _ROW, _ROW)], tabv)
    for k in range(_NSLOT - 1):
        pltpu.async_copy(
            p_hbm.at[pl.ds((k * _NW + wid) * _ROW, _ROW)], bufs[k], gsems[k]
        )

    def group(g, _):
        for k in range(_NSLOT):
            t = g * _NSLOT + k
            pslot = (k + _NSLOT - 1) % _NSLOT
            tf = t + _NSLOT - 1

            @pl.when(jnp.logical_and(t >= 1, tf < _B))
            def _wait_prev_scatter():
                pltpu.make_async_copy(
                    bufs[pslot], o_hbm.at[pl.ds(0, _ROW)], ssems[pslot]
                ).wait()

            @pl.when(tf < _B)
            def _prefetch():
                pltpu.async_copy(
                    p_hbm.at[pl.ds((tf * _NW + wid) * _ROW, _ROW)],
                    bufs[pslot], gsems[pslot],
                )

            pltpu.make_async_copy(
                p_hbm.at[pl.ds(0, _ROW)], bufs[k], gsems[k]
            ).wait()

            _add_table(bufs[k], tabv)

            pltpu.async_copy(
                bufs[k], o_hbm.at[pl.ds((t * _NW + wid) * _ROW, _ROW)], ssems[k]
            )

        return 0

    lax.fori_loop(0, _B // _NSLOT, group, 0)
    for k in range(_NSLOT):
        pltpu.make_async_copy(
            bufs[k], o_hbm.at[pl.ds(0, _ROW)], ssems[k]
        ).wait()


def kernel(patch, pos_table):
    B, P, D = patch.shape
    patch1 = patch.reshape(_TOTAL)
    table1 = pos_table.reshape(_NW * _ROW)
    mesh = plsc.VectorSubcoreMesh(core_axis_name="c", subcore_axis_name="s")
    scratch = (
        [pltpu.VMEM((_ROW,), jnp.float32)]
        + [pltpu.VMEM((_ROW,), jnp.float32) for _ in range(_NSLOT)]
        + [pltpu.SemaphoreType.DMA for _ in range(2 * _NSLOT)]
    )
    f = pl.kernel(
        _sc_body,
        out_type=jax.ShapeDtypeStruct((_TOTAL,), jnp.float32),
        mesh=mesh,
        scratch_types=scratch,
    )
    out = f(patch1, table1)
    return out.reshape(B, P, D)


# speedup vs baseline: 1.0034x; 1.0034x over previous
import jax
import jax.numpy as jnp
from jax import lax
from jax.experimental import pallas as pl
from jax.experimental.pallas import tpu as pltpu
from jax.experimental.pallas import tpu_sc as plsc

_ROW = 3456
_NW = 32
_B = 128
_NSLOT = 8
_UNROLL = 24
_VEC = 16
_TOTAL = _B * _NW * _ROW


def _add_table(buf, tabv):
    steps = _ROW // (_VEC * _UNROLL)

    def jbody(j, _):
        base = j * (_VEC * _UNROLL)
        for u in range(_UNROLL):
            i = base + u * _VEC
            buf[pl.ds(i, _VEC)] = buf[pl.ds(i, _VEC)] + tabv[pl.ds(i, _VEC)]
        return 0

    lax.fori_loop(0, steps, jbody, 0)


def _sc_body(p_hbm, t_hbm, o_hbm, tabv, *scr):
    bufs = scr[0:_NSLOT]
    gsems = scr[_NSLOT:2 * _NSLOT]
    ssems = scr[2 * _NSLOT:3 * _NSLOT]

    wid = lax.axis_index("s") * 2 + lax.axis_index("c")

    pltpu.sync_copy(t_hbm.at[pl.ds(wid * _ROW, _ROW)], tabv)
    for k in range(_NSLOT - 1):
        pltpu.async_copy(
            p_hbm.at[pl.ds((k * _NW + wid) * _ROW, _ROW)], bufs[k], gsems[k]
        )

    def group(g, _):
        for k in range(_NSLOT):
            t = g * _NSLOT + k
            pslot = (k + _NSLOT - 1) % _NSLOT
            tf = t + _NSLOT - 1

            @pl.when(jnp.logical_and(t >= 1, tf < _B))
            def _wait_prev_scatter():
                pltpu.make_async_copy(
                    bufs[pslot], o_hbm.at[pl.ds(0, _ROW)], ssems[pslot]
                ).wait()

            @pl.when(tf < _B)
            def _prefetch():
                pltpu.async_copy(
                    p_hbm.at[pl.ds((tf * _NW + wid) * _ROW, _ROW)],
                    bufs[pslot], gsems[pslot],
                )

            pltpu.make_async_copy(
                p_hbm.at[pl.ds(0, _ROW)], bufs[k], gsems[k]
            ).wait()

            _add_table(bufs[k], tabv)

            pltpu.async_copy(
                bufs[k], o_hbm.at[pl.ds((t * _NW + wid) * _ROW, _ROW)], ssems[k]
            )

        return 0

    lax.fori_loop(0, _B // _NSLOT, group, 0)
    for k in range(_NSLOT):
        pltpu.make_async_copy(
            bufs[k], o_hbm.at[pl.ds(0, _ROW)], ssems[k]
        ).wait()


def kernel(patch, pos_table):
    B, P, D = patch.shape
    patch1 = patch.reshape(_TOTAL)
    table1 = pos_table.reshape(_NW * _ROW)
    mesh = plsc.VectorSubcoreMesh(core_axis_name="c", subcore_axis_name="s")
    scratch = (
        [pltpu.VMEM((_ROW,), jnp.float32)]
        + [pltpu.VMEM((_ROW,), jnp.float32) for _ in range(_NSLOT)]
        + [pltpu.SemaphoreType.DMA for _ in range(2 * _NSLOT)]
    )
    f = pl.kernel(
        _sc_body,
        out_type=jax.ShapeDtypeStruct((_TOTAL,), jnp.float32),
        mesh=mesh,
        scratch_types=scratch,
        compiler_params=pltpu.CompilerParams(use_tc_tiling_on_sc=True),
    )
    out = f(patch1, table1)
    return out.reshape(B, P, D)


# speedup vs baseline: 1.0101x; 1.0067x over previous
import jax
import jax.numpy as jnp
from jax import lax
from jax.experimental import pallas as pl
from jax.experimental.pallas import tpu as pltpu
from jax.experimental.pallas import tpu_sc as plsc

_ROW = 3456
_NW = 32
_B = 128
_NSLOT = 8
_UNROLL = 24
_VEC = 16
_TOTAL = _B * _NW * _ROW


def _add_table(buf, tabv):
    steps = _ROW // (_VEC * _UNROLL)

    def jbody(j, _):
        base = j * (_VEC * _UNROLL)
        for u in range(_UNROLL):
            i = base + u * _VEC
            buf[pl.ds(i, _VEC)] = buf[pl.ds(i, _VEC)] + tabv[pl.ds(i, _VEC)]
        return 0

    lax.fori_loop(0, steps, jbody, 0)


def _sc_body(p_hbm, t_hbm, o_hbm, tabv, *scr):
    bufs = scr[0:_NSLOT]
    gsems = scr[_NSLOT:2 * _NSLOT]
    ssems = scr[2 * _NSLOT:3 * _NSLOT]

    wid = lax.axis_index("s") * 2 + lax.axis_index("c")

    pltpu.sync_copy(t_hbm.at[pl.ds(wid * _ROW, _ROW)], tabv)
    for k in range(_NSLOT - 1):
        pltpu.async_copy(
            p_hbm.at[pl.ds((k * _NW + wid) * _ROW, _ROW)], bufs[k], gsems[k]
        )

    def group(g, _):
        for k in range(_NSLOT):
            t = g * _NSLOT + k
            pslot = (k + _NSLOT - 1) % _NSLOT
            tf = t + _NSLOT - 1

            @pl.when(jnp.logical_and(t >= 1, tf < _B))
            def _wait_prev_scatter():
                pltpu.make_async_copy(
                    bufs[pslot], o_hbm.at[pl.ds(0, _ROW)], ssems[pslot]
                ).wait()

            @pl.when(tf < _B)
            def _prefetch():
                pltpu.async_copy(
                    p_hbm.at[pl.ds((tf * _NW + wid) * _ROW, _ROW)],
                    bufs[pslot], gsems[pslot],
                )

            pltpu.make_async_copy(
                p_hbm.at[pl.ds(0, _ROW)], bufs[k], gsems[k]
            ).wait()

            _add_table(bufs[k], tabv)

            pltpu.async_copy(
                bufs[k], o_hbm.at[pl.ds((t * _NW + wid) * _ROW, _ROW)], ssems[k]
            )

        return 0

    lax.fori_loop(0, _B // _NSLOT, group, 0)
    for k in range(_NSLOT):
        pltpu.make_async_copy(
            bufs[k], o_hbm.at[pl.ds(0, _ROW)], ssems[k]
        ).wait()


def kernel(patch, pos_table):
    B, P, D = patch.shape
    patch1 = patch.reshape(_TOTAL)
    table1 = pos_table.reshape(_NW * _ROW)
    mesh = plsc.VectorSubcoreMesh(core_axis_name="c", subcore_axis_name="s")
    scratch = (
        [pltpu.VMEM((_ROW,), jnp.float32)]
        + [pltpu.VMEM((_ROW,), jnp.float32) for _ in range(_NSLOT)]
        + [pltpu.SemaphoreType.DMA for _ in range(2 * _NSLOT)]
    )
    f = pl.kernel(
        _sc_body,
        out_type=jax.ShapeDtypeStruct((_TOTAL,), jnp.float32),
        mesh=mesh,
        scratch_types=scratch,
    )
    out = f(patch1, table1)
    return out.reshape(B, P, D)
